# baseline (speedup 1.0000x reference)
import functools

import jax
import jax.numpy as jnp
from jax import lax
from jax.experimental import pallas as pl
from jax.experimental.pallas import tpu as pltpu
from jax.experimental.pallas import tpu_sc as plsc

NW = 32


def _emb_call(B, H, D, V):
    NB = B // 128
    assert NB == NW and D == 64 and H % 2 == 0
    mesh = plsc.VectorSubcoreMesh(core_axis_name="c", subcore_axis_name="s")

    @functools.partial(
        pl.kernel,
        mesh=mesh,
        out_type=jax.ShapeDtypeStruct((H, 8, NB, 8, 128), jnp.float32),
        scratch_types=[
            pltpu.VMEM((H, 128), jnp.int32),
            pltpu.VMEM((H, 128), jnp.int32),
            pltpu.VMEM((2, 128, 128), jnp.float32),
            pltpu.VMEM((2, 8, 8, 128), jnp.float32),
            pltpu.SemaphoreType.DMA,
            pltpu.SemaphoreType.DMA,
            pltpu.SemaphoreType.DMA,
            pltpu.SemaphoreType.DMA,
        ],
        compiler_params=pltpu.CompilerParams(
            use_tc_tiling_on_sc=True, needs_layout_passes=False),
    )
    def emb(tp_hbm, xt_hbm, out_hbm, half_v, pair_v, pairbuf, trbuf,
            g0, g1, w0, w1):
        w = lax.axis_index("s") * 2 + lax.axis_index("c")
        gsems = (g0, g1)
        wsems = (w0, w1)
        iota = lax.iota(jnp.int32, 16)

        pltpu.sync_copy(xt_hbm.at[:, pl.ds(w * 128, 128)], half_v)

        def prep(h, c):
            for g in range(8):
                v = half_v[h, pl.ds(g * 16, 16)]
                pair_v[h, pl.ds(g * 16, 16)] = lax.shift_right_logical(v, 1)
                half_v[h, pl.ds(g * 16, 16)] = lax.shift_left(
                    lax.bitwise_and(v, 1), 6)
            return c

        lax.fori_loop(0, H, prep, 0)

        def fire_g(h, s):
            pltpu.async_copy(tp_hbm.at[pair_v.at[h]], pairbuf.at[s], gsems[s])

        def drain_g(h, s):
            pltpu.make_async_copy(
                tp_hbm.at[pair_v.at[h]], pairbuf.at[s], gsems[s]).wait()

        def fire_w(h, s):
            pltpu.async_copy(trbuf.at[s], out_hbm.at[h, :, w], wsems[s])

        def wait_w(h, s):
            pltpu.make_async_copy(
                trbuf.at[s], out_hbm.at[h, :, w], wsems[s]).wait()

        def transpose(h, s):
            for g in range(8):
                rowv = iota + (g * 16)
                halfv = half_v[h, pl.ds(g * 16, 16)]
                for f in range(64):
                    vals = plsc.load_gather(
                        pairbuf.at[s], [rowv, halfv + f])
                    trbuf[s, f // 8, f % 8, pl.ds(g * 16, 16)] = vals

        fire_g(0, 0)
        fire_g(1, 1)

        def body(i, c):
            hh = i * 2
            for b in range(2):
                h = hh + b
                drain_g(h, b)
                transpose(h, b)
                fire_w(h, b)
                wait_w(h, b)
                fire_g(h + 2, b)
            return c

        lax.fori_loop(0, (H - 2) // 2, body, 0)

        for b in range(2):
            h = H - 2 + b
            drain_g(h, b)
            transpose(h, b)
            fire_w(h, b)
        for b in range(2):
            wait_w(H - 2 + b, b)

    return emb


def kernel(x, table):
    B, H = x.shape
    V, D = table.shape
    tpairs = table.reshape(V // 2, 2 * D)
    xt = jnp.transpose(x).astype(jnp.int32)
    out5 = _emb_call(B, H, D, V)(tpairs, xt)
    return out5.transpose(2, 4, 0, 1, 3).reshape(B, H, D)


# speedup vs baseline: 1.0486x; 1.0486x over previous
import functools

import jax
import jax.numpy as jnp
from jax import lax
from jax.experimental import pallas as pl
from jax.experimental.pallas import tpu as pltpu
from jax.experimental.pallas import tpu_sc as plsc

NW = 32


def _emb_call(B, H, D, V):
    NB = B // 128
    assert NB == NW and D == 64 and H % 2 == 0
    mesh = plsc.VectorSubcoreMesh(core_axis_name="c", subcore_axis_name="s")

    @functools.partial(
        pl.kernel,
        mesh=mesh,
        out_type=jax.ShapeDtypeStruct((H, 8, NB, 8, 128), jnp.float32),
        scratch_types=[
            pltpu.VMEM((H, 128), jnp.int32),
            pltpu.VMEM((4, 128), jnp.int32),
            pltpu.VMEM((4, 128, 128), jnp.float32),
            pltpu.VMEM((4, 8, 8, 128), jnp.float32),
            pltpu.SemaphoreType.DMA,
            pltpu.SemaphoreType.DMA,
            pltpu.SemaphoreType.DMA,
            pltpu.SemaphoreType.DMA,
            pltpu.SemaphoreType.DMA,
            pltpu.SemaphoreType.DMA,
            pltpu.SemaphoreType.DMA,
            pltpu.SemaphoreType.DMA,
        ],
        compiler_params=pltpu.CompilerParams(
            use_tc_tiling_on_sc=True, needs_layout_passes=False),
    )
    def emb(tp_hbm, xt_hbm, out_hbm, idx_v, pid_v, pairbuf, trbuf,
            g0, g1, g2, g3, w0, w1, w2, w3):
        w = lax.axis_index("s") * 2 + lax.axis_index("c")
        gsems = (g0, g1, g2, g3)
        wsems = (w0, w1, w2, w3)
        iota = lax.iota(jnp.int32, 16)

        pltpu.sync_copy(xt_hbm.at[:, pl.ds(w * 128, 128)], idx_v)

        def prep(h, s):
            for g in range(8):
                v = idx_v[h, pl.ds(g * 16, 16)]
                pid_v[s, pl.ds(g * 16, 16)] = lax.shift_right_logical(v, 1)

        def fire_g(s):
            pltpu.async_copy(tp_hbm.at[pid_v.at[s]], pairbuf.at[s], gsems[s])

        def drain_g(s):
            pltpu.make_async_copy(
                tp_hbm.at[pid_v.at[s]], pairbuf.at[s], gsems[s]).wait()

        def fire_w(h, s):
            pltpu.async_copy(trbuf.at[s], out_hbm.at[h, :, w], wsems[s])

        def wait_w(h, s):
            pltpu.make_async_copy(
                trbuf.at[s], out_hbm.at[h, :, w], wsems[s]).wait()

        def transpose(h, s):
            def tgroup(g, c):
                rowv = iota + g * 16
                v = idx_v[h, pl.ds(g * 16, 16)]
                halfv = lax.shift_left(lax.bitwise_and(v, 1), 6)
                for f in range(64):
                    vals = plsc.load_gather(
                        pairbuf.at[s], [rowv, halfv + f])
                    trbuf[s, f // 8, f % 8, pl.ds(g * 16, 16)] = vals
                return c

            lax.fori_loop(0, 8, tgroup, 0)

        for s in range(4):
            prep(s, s)
            fire_g(s)

        for s in range(4):
            drain_g(s)
            transpose(s, s)
            fire_w(s, s)
            prep(s + 4, s)
            fire_g(s)

        def body(i, c):
            hh = i * 4
            for s in range(4):
                h = hh + s
                drain_g(s)
                wait_w(h - 4, s)
                transpose(h, s)
                fire_w(h, s)
                prep(h + 4, s)
                fire_g(s)
            return c

        lax.fori_loop(1, (H - 4) // 4, body, 0)

        for s in range(4):
            h = H - 4 + s
            drain_g(s)
            wait_w(h - 4, s)
            transpose(h, s)
            fire_w(h, s)
        for s in range(4):
            wait_w(H - 4 + s, s)

    return emb


def kernel(x, table):
    B, H = x.shape
    V, D = table.shape
    tpairs = table.reshape(V // 2, 2 * D)
    xt = jnp.transpose(x).astype(jnp.int32)
    out5 = _emb_call(B, H, D, V)(tpairs, xt)
    return out5.transpose(2, 4, 0, 1, 3).reshape(B, H, D)


# speedup vs baseline: 1.5696x; 1.4969x over previous
import functools

import jax
import jax.numpy as jnp
from jax import lax
from jax.experimental import pallas as pl
from jax.experimental.pallas import tpu as pltpu
from jax.experimental.pallas import tpu_sc as plsc

NW = 32
NR = 4


def _emb_call(B, H, D, rpw):
    nch = rpw // NR
    splits = []
    off = 0
    while off < H:
        w = min(128, H - off)
        splits.append((off, w))
        off += w
    mesh = plsc.VectorSubcoreMesh(core_axis_name="c", subcore_axis_name="s")

    @functools.partial(
        pl.kernel,
        mesh=mesh,
        out_type=jax.ShapeDtypeStruct((B, H, D), jnp.float32),
        scratch_types=[
            pltpu.VMEM((rpw, H), jnp.int32),
            pltpu.VMEM((2, NR, H, D), jnp.float32),
            pltpu.SemaphoreType.DMA,
            pltpu.SemaphoreType.DMA,
            pltpu.SemaphoreType.DMA,
            pltpu.SemaphoreType.DMA,
        ],
        compiler_params=pltpu.CompilerParams(use_tc_tiling_on_sc=False),
    )
    def emb(table_hbm, idx_hbm, out_hbm, idx_v, rows_v, g0sem, g1sem,
            w0sem, w1sem):
        wid = lax.axis_index("s") * 2 + lax.axis_index("c")
        base = wid * rpw
        gsems = (g0sem, g1sem)
        wsems = (w0sem, w1sem)

        def fire_gathers(c, s):
            for r in range(NR):
                for (off, w) in splits:
                    pltpu.async_copy(
                        table_hbm.at[idx_v.at[c * NR + r, pl.ds(off, w)]],
                        rows_v.at[s].at[r].at[pl.ds(off, w)],
                        gsems[s],
                    )

        def drain_gathers(c, s):
            for r in range(NR):
                for (off, w) in splits:
                    pltpu.make_async_copy(
                        table_hbm.at[idx_v.at[c * NR + r, pl.ds(off, w)]],
                        rows_v.at[s].at[r].at[pl.ds(off, w)],
                        gsems[s],
                    ).wait()

        def fire_write(c, s):
            pltpu.async_copy(
                rows_v.at[s], out_hbm.at[pl.ds(base + c * NR, NR)], wsems[s])

        def wait_write(c, s):
            pltpu.make_async_copy(
                rows_v.at[s], out_hbm.at[pl.ds(base + c * NR, NR)],
                wsems[s]).wait()

        pltpu.sync_copy(idx_hbm.at[pl.ds(base, rpw)], idx_v)
        fire_gathers(0, 0)
        fire_gathers(1, 1)

        def body(i, carry):
            cc = i * 2
            for b in range(2):
                c = cc + b
                drain_gathers(c, b)
                fire_write(c, b)
                wait_write(c, b)
                fire_gathers(c + 2, b)
            return carry

        lax.fori_loop(0, (nch - 2) // 2, body, 0)

        for b in range(2):
            c = nch - 2 + b
            drain_gathers(c, b)
            fire_write(c, b)
        for b in range(2):
            wait_write(nch - 2 + b, b)

    return emb


def kernel(x, table):
    B, H = x.shape
    V, D = table.shape
    rpw = B // NW
    return _emb_call(B, H, D, rpw)(table, x.astype(jnp.int32))
